# baseline (speedup 1.0000x reference)
import jax
import jax.numpy as jnp
from jax.experimental import pallas as pl
from jax.experimental.pallas import tpu as pltpu


def _identity_dma_kernel(x_ref, o_ref, sem):
    copy = pltpu.make_async_copy(x_ref, o_ref, sem)
    copy.start()
    copy.wait()


def kernel(x, edge_index):
    return pl.pallas_call(
        _identity_dma_kernel,
        in_specs=[pl.BlockSpec(memory_space=pl.ANY)],
        out_specs=pl.BlockSpec(memory_space=pl.ANY),
        out_shape=jax.ShapeDtypeStruct(x.shape, x.dtype),
        scratch_shapes=[pltpu.SemaphoreType.DMA],
    )(x)


# speedup vs baseline: 30.2027x; 30.2027x over previous
import jax
import jax.numpy as jnp
from jax.experimental import pallas as pl
from jax.experimental.pallas import tpu as pltpu

_CHUNK = 2000


def _make_dbuf_kernel(n_chunks):
    def _dbuf_kernel(x_hbm, o_hbm, buf, in_sems, out_sems):
        def in_copy(i, slot):
            return pltpu.make_async_copy(
                x_hbm.at[pl.ds(i * _CHUNK, _CHUNK), :], buf.at[slot],
                in_sems.at[slot])

        def out_copy(i, slot):
            return pltpu.make_async_copy(
                buf.at[slot], o_hbm.at[pl.ds(i * _CHUNK, _CHUNK), :],
                out_sems.at[slot])

        in_copy(0, 0).start()

        def body(i, carry):
            slot = jax.lax.rem(i, 2)
            nxt = jax.lax.rem(i + 1, 2)

            @pl.when(i + 1 < n_chunks)
            def _():
                @pl.when(i >= 1)
                def _():
                    out_copy(i - 1, nxt).wait()
                in_copy(i + 1, nxt).start()

            in_copy(i, slot).wait()
            out_copy(i, slot).start()
            return carry

        jax.lax.fori_loop(0, n_chunks, body, 0)
        if n_chunks >= 2:
            out_copy(n_chunks - 2, (n_chunks - 2) % 2).wait()
        out_copy(n_chunks - 1, (n_chunks - 1) % 2).wait()

    return _dbuf_kernel


def kernel(x, edge_index):
    n, d = x.shape
    n_chunks = n // _CHUNK
    return pl.pallas_call(
        _make_dbuf_kernel(n_chunks),
        in_specs=[pl.BlockSpec(memory_space=pl.ANY)],
        out_specs=pl.BlockSpec(memory_space=pl.ANY),
        out_shape=jax.ShapeDtypeStruct((n, d), x.dtype),
        scratch_shapes=[
            pltpu.VMEM((2, _CHUNK, d), x.dtype),
            pltpu.SemaphoreType.DMA((2,)),
            pltpu.SemaphoreType.DMA((2,)),
        ],
    )(x)


# speedup vs baseline: 43.7504x; 1.4486x over previous
import jax
import jax.numpy as jnp
from jax.experimental import pallas as pl
from jax.experimental.pallas import tpu as pltpu

_CHUNK = 1000


def _make_copy_kernel(n_chunks):
    def _copy_kernel(x_hbm, o_hbm, buf, in_sems, out_sems):
        def in_copy(c):
            sl = pl.ds(c * _CHUNK, _CHUNK)
            return pltpu.make_async_copy(
                x_hbm.at[sl, :], buf.at[sl, :], in_sems.at[c])

        def out_copy(c):
            sl = pl.ds(c * _CHUNK, _CHUNK)
            return pltpu.make_async_copy(
                buf.at[sl, :], o_hbm.at[sl, :], out_sems.at[c])

        for c in range(n_chunks):
            in_copy(c).start()
        for c in range(n_chunks):
            in_copy(c).wait()
            out_copy(c).start()
        for c in range(n_chunks):
            out_copy(c).wait()

    return _copy_kernel


def kernel(x, edge_index):
    n, d = x.shape
    n_chunks = n // _CHUNK
    return pl.pallas_call(
        _make_copy_kernel(n_chunks),
        in_specs=[pl.BlockSpec(memory_space=pl.ANY)],
        out_specs=pl.BlockSpec(memory_space=pl.ANY),
        out_shape=jax.ShapeDtypeStruct((n, d), x.dtype),
        scratch_shapes=[
            pltpu.VMEM((n, d), x.dtype),
            pltpu.SemaphoreType.DMA((n_chunks,)),
            pltpu.SemaphoreType.DMA((n_chunks,)),
        ],
    )(x)
